# baseline (speedup 1.0000x reference)
import functools

import jax
import jax.numpy as jnp
from jax import lax
from jax.experimental import pallas as pl
from jax.experimental.pallas import tpu as pltpu
from jax.experimental.pallas import tpu_sc as plsc

N = 50000
IN_DIM = 128
HID = 64
NUM_LAYERS = 5

CHUNK = 128
TILES = 16
CPT = 400
IB = 80
NBLK = CPT // IB
NCHUNKS = TILES * CPT
EPAD = NCHUNKS * CHUNK
SROWS = 50048
ZR = SROWS // TILES
WR = ZR
DEGC = 8

RB = 2000
GRID = N // RB


def _sc_mesh():
    return plsc.VectorSubcoreMesh(core_axis_name="c", subcore_axis_name="s")


def _sc_degrees(src_d, dst_g, zeros_deg, ones_deg):
    @functools.partial(
        pl.kernel,
        mesh=_sc_mesh(),
        compiler_params=pltpu.CompilerParams(use_tc_tiling_on_sc=False),
        out_type=[
            jax.ShapeDtypeStruct((SROWS, DEGC), jnp.float32),
            jax.ShapeDtypeStruct((SROWS, DEGC), jnp.float32),
        ],
        scratch_types=[
            pltpu.VMEM((IB, CHUNK), jnp.int32),
            pltpu.VMEM((CHUNK, DEGC), jnp.float32),
            pltpu.VMEM_SHARED((SROWS, DEGC), jnp.float32),
            pltpu.SemaphoreType.DMA,
        ],
    )
    def deg_kernel(srcd_hbm, dstg_hbm, zeros_hbm, ones_hbm,
                   dego_hbm, degi_hbm, idx_v, ones_v, deg_sh, dsem):
        cid = lax.axis_index("c")
        sid = lax.axis_index("s")
        pltpu.sync_copy(zeros_hbm, deg_sh.at[pl.ds(sid * ZR, ZR)])
        pltpu.sync_copy(ones_hbm, ones_v)
        plsc.subcore_barrier()

        def blk(b, carry):
            off = sid * CPT + b * IB

            @pl.when(cid == 0)
            def _():
                pltpu.sync_copy(srcd_hbm.at[pl.ds(off, IB)], idx_v)

            @pl.when(cid == 1)
            def _():
                pltpu.sync_copy(dstg_hbm.at[pl.ds(off, IB)], idx_v)

            def s_start(j):
                pltpu.async_copy(ones_v, deg_sh.at[idx_v.at[j]], dsem,
                                 add=True)

            def s_wait(j):
                pltpu.make_async_copy(ones_v, deg_sh.at[idx_v.at[j]],
                                      dsem).wait()

            for j in range(4):
                s_start(j)

            def body(k, c):
                s_start(k + 4)
                s_wait(k)
                return c

            lax.fori_loop(0, IB - 4, body, 0)
            for j in range(4):
                s_wait(IB - 4 + j)
            return carry

        lax.fori_loop(0, NBLK, blk, 0)
        plsc.subcore_barrier()

        @pl.when(cid == 0)
        def _():
            pltpu.sync_copy(deg_sh.at[pl.ds(sid * WR, WR)],
                            dego_hbm.at[pl.ds(sid * WR, WR)])

        @pl.when(cid == 1)
        def _():
            pltpu.sync_copy(deg_sh.at[pl.ds(sid * WR, WR)],
                            degi_hbm.at[pl.ds(sid * WR, WR)])

    return deg_kernel(src_d, dst_g, zeros_deg, ones_deg)


def _sc_aggregate(f0, f1, src_g, dst_g, zeros_agg):
    @functools.partial(
        pl.kernel,
        mesh=_sc_mesh(),
        compiler_params=pltpu.CompilerParams(use_tc_tiling_on_sc=False),
        out_type=[
            jax.ShapeDtypeStruct((SROWS, HID // 2), jnp.float32),
            jax.ShapeDtypeStruct((SROWS, HID // 2), jnp.float32),
        ],
        scratch_types=[
            pltpu.VMEM((IB, CHUNK), jnp.int32),
            pltpu.VMEM((IB, CHUNK), jnp.int32),
            pltpu.VMEM((CHUNK, HID // 2), jnp.float32),
            pltpu.VMEM((CHUNK, HID // 2), jnp.float32),
            pltpu.VMEM_SHARED((SROWS, HID // 2), jnp.float32),
            pltpu.SemaphoreType.DMA,
            pltpu.SemaphoreType.DMA,
            pltpu.SemaphoreType.DMA,
            pltpu.SemaphoreType.DMA,
        ],
    )
    def agg_kernel(f0_hbm, f1_hbm, srcg_hbm, dstg_hbm, zeros_hbm,
                   out0_hbm, out1_hbm, src_v, dst_v, rows_a, rows_b,
                   agg_sh, sem_a, sem_b, ssem_a, ssem_b):
        cid = lax.axis_index("c")
        sid = lax.axis_index("s")
        pltpu.sync_copy(zeros_hbm, agg_sh.at[pl.ds(sid * ZR, ZR)])
        plsc.subcore_barrier()

        def gather_start(j, buf, sem):
            @pl.when(cid == 0)
            def _():
                pltpu.async_copy(f0_hbm.at[src_v.at[j]], buf, sem)

            @pl.when(cid == 1)
            def _():
                pltpu.async_copy(f1_hbm.at[src_v.at[j]], buf, sem)

        def gather_wait(j, buf, sem):
            pltpu.make_async_copy(f0_hbm.at[src_v.at[j]], buf, sem).wait()

        def scatter_start(j, buf, sem):
            pltpu.async_copy(buf, agg_sh.at[dst_v.at[j]], sem, add=True)

        def scatter_wait(j, buf, sem):
            pltpu.make_async_copy(buf, agg_sh.at[dst_v.at[j]], sem).wait()

        def blk(b, carry):
            off = sid * CPT + b * IB
            pltpu.sync_copy(srcg_hbm.at[pl.ds(off, IB)], src_v)
            pltpu.sync_copy(dstg_hbm.at[pl.ds(off, IB)], dst_v)
            gather_start(0, rows_a, sem_a)
            gather_start(1, rows_b, sem_b)

            def body(k2, c):
                j = 2 * k2
                gather_wait(j, rows_a, sem_a)
                scatter_start(j, rows_a, ssem_a)
                gather_wait(j + 1, rows_b, sem_b)
                scatter_start(j + 1, rows_b, ssem_b)
                scatter_wait(j, rows_a, ssem_a)

                @pl.when(j + 2 < IB)
                def _():
                    gather_start(j + 2, rows_a, sem_a)

                scatter_wait(j + 1, rows_b, ssem_b)

                @pl.when(j + 3 < IB)
                def _():
                    gather_start(j + 3, rows_b, sem_b)

                return c

            lax.fori_loop(0, IB // 2, body, 0)
            return carry

        lax.fori_loop(0, NBLK, blk, 0)
        plsc.subcore_barrier()

        @pl.when(cid == 0)
        def _():
            pltpu.sync_copy(agg_sh.at[pl.ds(sid * WR, WR)],
                            out0_hbm.at[pl.ds(sid * WR, WR)])

        @pl.when(cid == 1)
        def _():
            pltpu.sync_copy(agg_sh.at[pl.ds(sid * WR, WR)],
                            out1_hbm.at[pl.ds(sid * WR, WR)])

    return agg_kernel(f0, f1, src_g, dst_g, zeros_agg)


def _tc_enc_a(h, enc_W1, b1):
    def body(h_ref, w_ref, b_ref, t_ref, stats_ref, acc_ref):
        i = pl.program_id(0)
        t = lax.dot_general(h_ref[...], w_ref[...], (((1,), (1,)), ((), ())),
                            preferred_element_type=jnp.float32) + b_ref[...]
        t_ref[...] = t
        s = jnp.concatenate([jnp.sum(t, 0, keepdims=True),
                             jnp.sum(t * t, 0, keepdims=True)], 0)

        @pl.when(i == 0)
        def _():
            acc_ref[...] = jnp.zeros_like(acc_ref)

        acc_ref[...] += s
        stats_ref[...] = acc_ref[...]

    return pl.pallas_call(
        body,
        grid=(GRID,),
        in_specs=[
            pl.BlockSpec((RB, IN_DIM), lambda i: (i, 0)),
            pl.BlockSpec((HID, IN_DIM), lambda i: (0, 0)),
            pl.BlockSpec((1, HID), lambda i: (0, 0)),
        ],
        out_specs=[
            pl.BlockSpec((RB, HID), lambda i: (i, 0)),
            pl.BlockSpec((2, HID), lambda i: (0, 0)),
        ],
        out_shape=[
            jax.ShapeDtypeStruct((N, HID), jnp.float32),
            jax.ShapeDtypeStruct((2, HID), jnp.float32),
        ],
        scratch_shapes=[pltpu.VMEM((2, HID), jnp.float32)],
    )(h, enc_W1, b1)


def _tc_enc_b(t, stats, gamma, beta, enc_W2, b2, deg_s):
    def body(t_ref, st_ref, g_ref, be_ref, w_ref, b_ref, ds_ref,
             x_ref, f0_ref, f1_ref):
        s = st_ref[...]
        mean = s[0:1, :] * (1.0 / N)
        var = s[1:2, :] * (1.0 / N) - mean * mean
        inv = lax.rsqrt(var + 1e-5)
        xn = (t_ref[...] - mean) * (inv * g_ref[...]) + be_ref[...]
        xn = jnp.maximum(xn, 0.0)
        x = lax.dot_general(xn, w_ref[...], (((1,), (1,)), ((), ())),
                            preferred_element_type=jnp.float32) + b_ref[...]
        x_ref[...] = x
        ns = lax.rsqrt(jnp.maximum(ds_ref[...][:, 0:1], 1.0))
        f = x * ns
        f0_ref[...] = f[:, : HID // 2]
        f1_ref[...] = f[:, HID // 2:]

    return pl.pallas_call(
        body,
        grid=(GRID,),
        in_specs=[
            pl.BlockSpec((RB, HID), lambda i: (i, 0)),
            pl.BlockSpec((2, HID), lambda i: (0, 0)),
            pl.BlockSpec((1, HID), lambda i: (0, 0)),
            pl.BlockSpec((1, HID), lambda i: (0, 0)),
            pl.BlockSpec((HID, HID), lambda i: (0, 0)),
            pl.BlockSpec((1, HID), lambda i: (0, 0)),
            pl.BlockSpec((RB, DEGC), lambda i: (i, 0)),
        ],
        out_specs=[
            pl.BlockSpec((RB, HID), lambda i: (i, 0)),
            pl.BlockSpec((RB, HID // 2), lambda i: (i, 0)),
            pl.BlockSpec((RB, HID // 2), lambda i: (i, 0)),
        ],
        out_shape=[
            jax.ShapeDtypeStruct((N, HID), jnp.float32),
            jax.ShapeDtypeStruct((N, HID // 2), jnp.float32),
            jax.ShapeDtypeStruct((N, HID // 2), jnp.float32),
        ],
    )(t, stats, gamma, beta, enc_W2, b2, deg_s)


def _tc_layer(a0, a1, deg_d, deg_s, W, acc, last):
    def body(a0_ref, a1_ref, dd_ref, ds_ref, w_ref, acc_ref, *out_refs):
        agg = jnp.concatenate([a0_ref[...], a1_ref[...]], 1)
        nd = lax.rsqrt(jnp.maximum(dd_ref[...][:, 0:1], 1.0))
        hc = lax.dot_general(agg * nd, w_ref[...], (((1,), (0,)), ((), ())),
                             preferred_element_type=jnp.float32)
        if not last:
            hc = jnp.maximum(hc, 0.0)
            out_refs[0][...] = acc_ref[...] + hc
            ns = lax.rsqrt(jnp.maximum(ds_ref[...][:, 0:1], 1.0))
            f = hc * ns
            out_refs[1][...] = f[:, : HID // 2]
            out_refs[2][...] = f[:, HID // 2:]
        else:
            out_refs[0][...] = (acc_ref[...] + hc) * (1.0 / (NUM_LAYERS + 1))

    out_specs = [pl.BlockSpec((RB, HID), lambda i: (i, 0))]
    out_shape = [jax.ShapeDtypeStruct((N, HID), jnp.float32)]
    if not last:
        out_specs += [pl.BlockSpec((RB, HID // 2), lambda i: (i, 0))] * 2
        out_shape += [jax.ShapeDtypeStruct((N, HID // 2), jnp.float32)] * 2

    return pl.pallas_call(
        body,
        grid=(GRID,),
        in_specs=[
            pl.BlockSpec((RB, HID // 2), lambda i: (i, 0)),
            pl.BlockSpec((RB, HID // 2), lambda i: (i, 0)),
            pl.BlockSpec((RB, DEGC), lambda i: (i, 0)),
            pl.BlockSpec((RB, DEGC), lambda i: (i, 0)),
            pl.BlockSpec((HID, HID), lambda i: (0, 0)),
            pl.BlockSpec((RB, HID), lambda i: (i, 0)),
        ],
        out_specs=out_specs,
        out_shape=out_shape,
    )(a0, a1, deg_d, deg_s, W, acc)


def kernel(h, edge_index, enc_W1, enc_b1, bn_gamma, bn_beta, enc_W2,
           enc_b2, gcn_W):
    E = edge_index.shape[1]
    pad = EPAD - E
    src = edge_index[0]
    dst = edge_index[1]
    zero_pad = jnp.zeros((pad,), jnp.int32)
    sac_pad = jnp.full((pad,), N, jnp.int32)
    src_g = jnp.concatenate([src, zero_pad]).reshape(NCHUNKS, CHUNK)
    dst_g = jnp.concatenate([dst, sac_pad]).reshape(NCHUNKS, CHUNK)
    src_d = jnp.concatenate([src, sac_pad]).reshape(NCHUNKS, CHUNK)

    zeros_deg = jnp.zeros((ZR, DEGC), jnp.float32)
    ones_deg = jnp.ones((CHUNK, DEGC), jnp.float32)
    zeros_agg = jnp.zeros((ZR, HID // 2), jnp.float32)

    deg_s, deg_d = _sc_degrees(src_d, dst_g, zeros_deg, ones_deg)
    deg_s = deg_s[:N]
    deg_d = deg_d[:N]

    b1 = enc_b1.reshape(1, HID)
    b2 = enc_b2.reshape(1, HID)
    gamma = bn_gamma.reshape(1, HID)
    beta = bn_beta.reshape(1, HID)

    t, stats = _tc_enc_a(h, enc_W1, b1)
    acc, f0, f1 = _tc_enc_b(t, stats, gamma, beta, enc_W2, b2, deg_s)

    for i in range(NUM_LAYERS):
        a0, a1 = _sc_aggregate(f0, f1, src_g, dst_g, zeros_agg)
        a0 = a0[:N]
        a1 = a1[:N]
        last = i == NUM_LAYERS - 1
        outs = _tc_layer(a0, a1, deg_d, deg_s, gcn_W[i], acc, last)
        if last:
            acc = outs[0]
        else:
            acc, f0, f1 = outs
    return acc


# speedup vs baseline: 1.0754x; 1.0754x over previous
import functools

import jax
import jax.numpy as jnp
from jax import lax
from jax.experimental import pallas as pl
from jax.experimental.pallas import tpu as pltpu
from jax.experimental.pallas import tpu_sc as plsc

N = 50000
IN_DIM = 128
HID = 64
NUM_LAYERS = 5

CHUNK = 128
TILES = 16
CPT = 400
IB = 80
NBLK = CPT // IB
NCHUNKS = TILES * CPT
EPAD = NCHUNKS * CHUNK
SROWS = 50048
ZR = SROWS // TILES
WR = ZR
DEGC = 8

RB = 2000
GRID = N // RB


def _sc_mesh():
    return plsc.VectorSubcoreMesh(core_axis_name="c", subcore_axis_name="s")


def _sc_degrees(src_d, dst_g, zeros_deg, ones_deg):
    @functools.partial(
        pl.kernel,
        mesh=_sc_mesh(),
        compiler_params=pltpu.CompilerParams(use_tc_tiling_on_sc=False),
        out_type=[
            jax.ShapeDtypeStruct((SROWS, DEGC), jnp.float32),
            jax.ShapeDtypeStruct((SROWS, DEGC), jnp.float32),
        ],
        scratch_types=[
            pltpu.VMEM((IB, CHUNK), jnp.int32),
            pltpu.VMEM((CHUNK, DEGC), jnp.float32),
            pltpu.VMEM_SHARED((SROWS, DEGC), jnp.float32),
            pltpu.SemaphoreType.DMA,
        ],
    )
    def deg_kernel(srcd_hbm, dstg_hbm, zeros_hbm, ones_hbm,
                   dego_hbm, degi_hbm, idx_v, ones_v, deg_sh, dsem):
        cid = lax.axis_index("c")
        sid = lax.axis_index("s")
        pltpu.sync_copy(zeros_hbm, deg_sh.at[pl.ds(sid * ZR, ZR)])
        pltpu.sync_copy(ones_hbm, ones_v)
        plsc.subcore_barrier()

        def blk(b, carry):
            off = sid * CPT + b * IB

            @pl.when(cid == 0)
            def _():
                pltpu.sync_copy(srcd_hbm.at[pl.ds(off, IB)], idx_v)

            @pl.when(cid == 1)
            def _():
                pltpu.sync_copy(dstg_hbm.at[pl.ds(off, IB)], idx_v)

            def s_start(j):
                pltpu.async_copy(ones_v, deg_sh.at[idx_v.at[j]], dsem,
                                 add=True)

            def s_wait(j):
                pltpu.make_async_copy(ones_v, deg_sh.at[idx_v.at[j]],
                                      dsem).wait()

            for j in range(4):
                s_start(j)

            def body(k, c):
                s_start(k + 4)
                s_wait(k)
                return c

            lax.fori_loop(0, IB - 4, body, 0)
            for j in range(4):
                s_wait(IB - 4 + j)
            return carry

        lax.fori_loop(0, NBLK, blk, 0)
        plsc.subcore_barrier()

        @pl.when(cid == 0)
        def _():
            pltpu.sync_copy(deg_sh.at[pl.ds(sid * WR, WR)],
                            dego_hbm.at[pl.ds(sid * WR, WR)])

        @pl.when(cid == 1)
        def _():
            pltpu.sync_copy(deg_sh.at[pl.ds(sid * WR, WR)],
                            degi_hbm.at[pl.ds(sid * WR, WR)])

    return deg_kernel(src_d, dst_g, zeros_deg, ones_deg)


def _sc_aggregate(f0, f1, src_g, dst_g, zeros_agg):
    @functools.partial(
        pl.kernel,
        mesh=_sc_mesh(),
        compiler_params=pltpu.CompilerParams(use_tc_tiling_on_sc=False),
        out_type=[
            jax.ShapeDtypeStruct((SROWS, HID // 2), jnp.float32),
            jax.ShapeDtypeStruct((SROWS, HID // 2), jnp.float32),
        ],
        scratch_types=[
            pltpu.VMEM((IB, CHUNK), jnp.int32),
            pltpu.VMEM((IB, CHUNK), jnp.int32),
            pltpu.VMEM((CHUNK, HID // 2), jnp.float32),
            pltpu.VMEM((CHUNK, HID // 2), jnp.float32),
            pltpu.VMEM_SHARED((SROWS, HID // 2), jnp.float32),
            pltpu.SemaphoreType.DMA,
            pltpu.SemaphoreType.DMA,
            pltpu.SemaphoreType.DMA,
            pltpu.SemaphoreType.DMA,
        ],
    )
    def agg_kernel(f0_hbm, f1_hbm, srcg_hbm, dstg_hbm, zeros_hbm,
                   out0_hbm, out1_hbm, src_v, dst_v, rows_a, rows_b,
                   agg_sh, sem_a, sem_b, ssem_a, ssem_b):
        cid = lax.axis_index("c")
        sid = lax.axis_index("s")
        pltpu.sync_copy(zeros_hbm, agg_sh.at[pl.ds(sid * ZR, ZR)])
        plsc.subcore_barrier()

        def gather_start(j, buf, sem):
            @pl.when(cid == 0)
            def _():
                pltpu.async_copy(f0_hbm.at[src_v.at[j]], buf, sem)

            @pl.when(cid == 1)
            def _():
                pltpu.async_copy(f1_hbm.at[src_v.at[j]], buf, sem)

        def gather_wait(j, buf, sem):
            pltpu.make_async_copy(f0_hbm.at[src_v.at[j]], buf, sem).wait()

        def scatter_start(j, buf, sem):
            pltpu.async_copy(buf, agg_sh.at[dst_v.at[j]], sem, add=True)

        def scatter_wait(j, buf, sem):
            pltpu.make_async_copy(buf, agg_sh.at[dst_v.at[j]], sem).wait()

        def blk(b, carry):
            off = sid * CPT + b * IB
            pltpu.sync_copy(srcg_hbm.at[pl.ds(off, IB)], src_v)
            pltpu.sync_copy(dstg_hbm.at[pl.ds(off, IB)], dst_v)
            gather_start(0, rows_a, sem_a)

            def body(k2, c):
                j = 2 * k2
                gather_start(j + 1, rows_b, sem_b)
                gather_wait(j, rows_a, sem_a)
                pltpu.sync_copy(rows_a, agg_sh.at[dst_v.at[j]], add=True)

                @pl.when(j + 2 < IB)
                def _():
                    gather_start(j + 2, rows_a, sem_a)

                gather_wait(j + 1, rows_b, sem_b)
                pltpu.sync_copy(rows_b, agg_sh.at[dst_v.at[j + 1]],
                                add=True)
                return c

            lax.fori_loop(0, IB // 2, body, 0)
            return carry

        lax.fori_loop(0, NBLK, blk, 0)
        plsc.subcore_barrier()

        @pl.when(cid == 0)
        def _():
            pltpu.sync_copy(agg_sh.at[pl.ds(sid * WR, WR)],
                            out0_hbm.at[pl.ds(sid * WR, WR)])

        @pl.when(cid == 1)
        def _():
            pltpu.sync_copy(agg_sh.at[pl.ds(sid * WR, WR)],
                            out1_hbm.at[pl.ds(sid * WR, WR)])

    return agg_kernel(f0, f1, src_g, dst_g, zeros_agg)


def _tc_enc_a(h, enc_W1, b1):
    def body(h_ref, w_ref, b_ref, t_ref, stats_ref, acc_ref):
        i = pl.program_id(0)
        t = lax.dot_general(h_ref[...], w_ref[...], (((1,), (1,)), ((), ())),
                            preferred_element_type=jnp.float32) + b_ref[...]
        t_ref[...] = t
        s = jnp.concatenate([jnp.sum(t, 0, keepdims=True),
                             jnp.sum(t * t, 0, keepdims=True)], 0)

        @pl.when(i == 0)
        def _():
            acc_ref[...] = jnp.zeros_like(acc_ref)

        acc_ref[...] += s
        stats_ref[...] = acc_ref[...]

    return pl.pallas_call(
        body,
        grid=(GRID,),
        in_specs=[
            pl.BlockSpec((RB, IN_DIM), lambda i: (i, 0)),
            pl.BlockSpec((HID, IN_DIM), lambda i: (0, 0)),
            pl.BlockSpec((1, HID), lambda i: (0, 0)),
        ],
        out_specs=[
            pl.BlockSpec((RB, HID), lambda i: (i, 0)),
            pl.BlockSpec((2, HID), lambda i: (0, 0)),
        ],
        out_shape=[
            jax.ShapeDtypeStruct((N, HID), jnp.float32),
            jax.ShapeDtypeStruct((2, HID), jnp.float32),
        ],
        scratch_shapes=[pltpu.VMEM((2, HID), jnp.float32)],
    )(h, enc_W1, b1)


def _tc_enc_b(t, stats, gamma, beta, enc_W2, b2, deg_s):
    def body(t_ref, st_ref, g_ref, be_ref, w_ref, b_ref, ds_ref,
             x_ref, f0_ref, f1_ref):
        s = st_ref[...]
        mean = s[0:1, :] * (1.0 / N)
        var = s[1:2, :] * (1.0 / N) - mean * mean
        inv = lax.rsqrt(var + 1e-5)
        xn = (t_ref[...] - mean) * (inv * g_ref[...]) + be_ref[...]
        xn = jnp.maximum(xn, 0.0)
        x = lax.dot_general(xn, w_ref[...], (((1,), (1,)), ((), ())),
                            preferred_element_type=jnp.float32) + b_ref[...]
        x_ref[...] = x
        ns = lax.rsqrt(jnp.maximum(ds_ref[...][:, 0:1], 1.0))
        f = x * ns
        f0_ref[...] = f[:, : HID // 2]
        f1_ref[...] = f[:, HID // 2:]

    return pl.pallas_call(
        body,
        grid=(GRID,),
        in_specs=[
            pl.BlockSpec((RB, HID), lambda i: (i, 0)),
            pl.BlockSpec((2, HID), lambda i: (0, 0)),
            pl.BlockSpec((1, HID), lambda i: (0, 0)),
            pl.BlockSpec((1, HID), lambda i: (0, 0)),
            pl.BlockSpec((HID, HID), lambda i: (0, 0)),
            pl.BlockSpec((1, HID), lambda i: (0, 0)),
            pl.BlockSpec((RB, DEGC), lambda i: (i, 0)),
        ],
        out_specs=[
            pl.BlockSpec((RB, HID), lambda i: (i, 0)),
            pl.BlockSpec((RB, HID // 2), lambda i: (i, 0)),
            pl.BlockSpec((RB, HID // 2), lambda i: (i, 0)),
        ],
        out_shape=[
            jax.ShapeDtypeStruct((N, HID), jnp.float32),
            jax.ShapeDtypeStruct((N, HID // 2), jnp.float32),
            jax.ShapeDtypeStruct((N, HID // 2), jnp.float32),
        ],
    )(t, stats, gamma, beta, enc_W2, b2, deg_s)


def _tc_layer(a0, a1, deg_d, deg_s, W, acc, last):
    def body(a0_ref, a1_ref, dd_ref, ds_ref, w_ref, acc_ref, *out_refs):
        agg = jnp.concatenate([a0_ref[...], a1_ref[...]], 1)
        nd = lax.rsqrt(jnp.maximum(dd_ref[...][:, 0:1], 1.0))
        hc = lax.dot_general(agg * nd, w_ref[...], (((1,), (0,)), ((), ())),
                             preferred_element_type=jnp.float32)
        if not last:
            hc = jnp.maximum(hc, 0.0)
            out_refs[0][...] = acc_ref[...] + hc
            ns = lax.rsqrt(jnp.maximum(ds_ref[...][:, 0:1], 1.0))
            f = hc * ns
            out_refs[1][...] = f[:, : HID // 2]
            out_refs[2][...] = f[:, HID // 2:]
        else:
            out_refs[0][...] = (acc_ref[...] + hc) * (1.0 / (NUM_LAYERS + 1))

    out_specs = [pl.BlockSpec((RB, HID), lambda i: (i, 0))]
    out_shape = [jax.ShapeDtypeStruct((N, HID), jnp.float32)]
    if not last:
        out_specs += [pl.BlockSpec((RB, HID // 2), lambda i: (i, 0))] * 2
        out_shape += [jax.ShapeDtypeStruct((N, HID // 2), jnp.float32)] * 2

    return pl.pallas_call(
        body,
        grid=(GRID,),
        in_specs=[
            pl.BlockSpec((RB, HID // 2), lambda i: (i, 0)),
            pl.BlockSpec((RB, HID // 2), lambda i: (i, 0)),
            pl.BlockSpec((RB, DEGC), lambda i: (i, 0)),
            pl.BlockSpec((RB, DEGC), lambda i: (i, 0)),
            pl.BlockSpec((HID, HID), lambda i: (0, 0)),
            pl.BlockSpec((RB, HID), lambda i: (i, 0)),
        ],
        out_specs=out_specs,
        out_shape=out_shape,
    )(a0, a1, deg_d, deg_s, W, acc)


def kernel(h, edge_index, enc_W1, enc_b1, bn_gamma, bn_beta, enc_W2,
           enc_b2, gcn_W):
    E = edge_index.shape[1]
    pad = EPAD - E
    src = edge_index[0]
    dst = edge_index[1]
    zero_pad = jnp.zeros((pad,), jnp.int32)
    sac_pad = jnp.full((pad,), N, jnp.int32)
    src_g = jnp.concatenate([src, zero_pad]).reshape(NCHUNKS, CHUNK)
    dst_g = jnp.concatenate([dst, sac_pad]).reshape(NCHUNKS, CHUNK)
    src_d = jnp.concatenate([src, sac_pad]).reshape(NCHUNKS, CHUNK)

    zeros_deg = jnp.zeros((ZR, DEGC), jnp.float32)
    ones_deg = jnp.ones((CHUNK, DEGC), jnp.float32)
    zeros_agg = jnp.zeros((ZR, HID // 2), jnp.float32)

    deg_s, deg_d = _sc_degrees(src_d, dst_g, zeros_deg, ones_deg)
    deg_s = deg_s[:N]
    deg_d = deg_d[:N]

    b1 = enc_b1.reshape(1, HID)
    b2 = enc_b2.reshape(1, HID)
    gamma = bn_gamma.reshape(1, HID)
    beta = bn_beta.reshape(1, HID)

    t, stats = _tc_enc_a(h, enc_W1, b1)
    acc, f0, f1 = _tc_enc_b(t, stats, gamma, beta, enc_W2, b2, deg_s)

    for i in range(NUM_LAYERS):
        a0, a1 = _sc_aggregate(f0, f1, src_g, dst_g, zeros_agg)
        a0 = a0[:N]
        a1 = a1[:N]
        last = i == NUM_LAYERS - 1
        outs = _tc_layer(a0, a1, deg_d, deg_s, gcn_W[i], acc, last)
        if last:
            acc = outs[0]
        else:
            acc, f0, f1 = outs
    return acc


# speedup vs baseline: 1.1575x; 1.0764x over previous
import functools

import jax
import jax.numpy as jnp
from jax import lax
from jax.experimental import pallas as pl
from jax.experimental.pallas import tpu as pltpu
from jax.experimental.pallas import tpu_sc as plsc

N = 50000
IN_DIM = 128
HID = 64
NUM_LAYERS = 5

CHUNK = 128
TILES = 16
CPT = 400
IB = 80
NBLK = CPT // IB
NCHUNKS = TILES * CPT
EPAD = NCHUNKS * CHUNK
SROWS = 50048
ZR = SROWS // TILES
WR = ZR
DEGC = 8

RB = 2000
GRID = N // RB


def _sc_mesh():
    return plsc.VectorSubcoreMesh(core_axis_name="c", subcore_axis_name="s")


def _sc_degrees(src_d, dst_g, zeros_deg, ones_deg):
    @functools.partial(
        pl.kernel,
        mesh=_sc_mesh(),
        compiler_params=pltpu.CompilerParams(use_tc_tiling_on_sc=False),
        out_type=[
            jax.ShapeDtypeStruct((SROWS, DEGC), jnp.float32),
            jax.ShapeDtypeStruct((SROWS, DEGC), jnp.float32),
        ],
        scratch_types=[
            pltpu.VMEM((IB, CHUNK), jnp.int32),
            pltpu.VMEM((CHUNK, DEGC), jnp.float32),
            pltpu.VMEM_SHARED((SROWS, DEGC), jnp.float32),
            pltpu.SemaphoreType.DMA,
        ],
    )
    def deg_kernel(srcd_hbm, dstg_hbm, zeros_hbm, ones_hbm,
                   dego_hbm, degi_hbm, idx_v, ones_v, deg_sh, dsem):
        cid = lax.axis_index("c")
        sid = lax.axis_index("s")
        pltpu.sync_copy(zeros_hbm, deg_sh.at[pl.ds(sid * ZR, ZR)])
        pltpu.sync_copy(ones_hbm, ones_v)
        plsc.subcore_barrier()

        def blk(b, carry):
            off = sid * CPT + b * IB

            @pl.when(cid == 0)
            def _():
                pltpu.sync_copy(srcd_hbm.at[pl.ds(off, IB)], idx_v)

            @pl.when(cid == 1)
            def _():
                pltpu.sync_copy(dstg_hbm.at[pl.ds(off, IB)], idx_v)

            def s_start(j):
                pltpu.async_copy(ones_v, deg_sh.at[idx_v.at[j]], dsem,
                                 add=True)

            def s_wait(j):
                pltpu.make_async_copy(ones_v, deg_sh.at[idx_v.at[j]],
                                      dsem).wait()

            for j in range(4):
                s_start(j)

            def body(k, c):
                s_start(k + 4)
                s_wait(k)
                return c

            lax.fori_loop(0, IB - 4, body, 0)
            for j in range(4):
                s_wait(IB - 4 + j)
            return carry

        lax.fori_loop(0, NBLK, blk, 0)
        plsc.subcore_barrier()

        @pl.when(cid == 0)
        def _():
            pltpu.sync_copy(deg_sh.at[pl.ds(sid * WR, WR)],
                            dego_hbm.at[pl.ds(sid * WR, WR)])

        @pl.when(cid == 1)
        def _():
            pltpu.sync_copy(deg_sh.at[pl.ds(sid * WR, WR)],
                            degi_hbm.at[pl.ds(sid * WR, WR)])

    return deg_kernel(src_d, dst_g, zeros_deg, ones_deg)


def _sc_aggregate(f0, f1, src_g, dst_g, zeros_agg):
    @functools.partial(
        pl.kernel,
        mesh=_sc_mesh(),
        compiler_params=pltpu.CompilerParams(use_tc_tiling_on_sc=False),
        out_type=[
            jax.ShapeDtypeStruct((SROWS, HID // 2), jnp.float32),
            jax.ShapeDtypeStruct((SROWS, HID // 2), jnp.float32),
        ],
        scratch_types=[
            pltpu.VMEM((IB, CHUNK), jnp.int32),
            pltpu.VMEM((IB, CHUNK), jnp.int32),
            pltpu.VMEM((CHUNK, HID // 2), jnp.float32),
            pltpu.VMEM((CHUNK, HID // 2), jnp.float32),
            pltpu.VMEM_SHARED((SROWS, HID // 2), jnp.float32),
            pltpu.SemaphoreType.DMA,
            pltpu.SemaphoreType.DMA,
            pltpu.SemaphoreType.DMA,
            pltpu.SemaphoreType.DMA,
        ],
    )
    def agg_kernel(f0_hbm, f1_hbm, srcg_hbm, dstg_hbm, zeros_hbm,
                   out0_hbm, out1_hbm, src_v, dst_v, rows_a, rows_b,
                   agg_sh, sem_a, sem_b, ssem_a, ssem_b):
        cid = lax.axis_index("c")
        sid = lax.axis_index("s")
        pltpu.sync_copy(zeros_hbm, agg_sh.at[pl.ds(sid * ZR, ZR)])
        plsc.subcore_barrier()

        def gather_start(j, buf, sem):
            @pl.when(cid == 0)
            def _():
                pltpu.async_copy(f0_hbm.at[src_v.at[j]], buf, sem)

            @pl.when(cid == 1)
            def _():
                pltpu.async_copy(f1_hbm.at[src_v.at[j]], buf, sem)

        def gather_wait(j, buf, sem):
            pltpu.make_async_copy(f0_hbm.at[src_v.at[j]], buf, sem).wait()

        def scatter_start(j, buf, sem):
            pltpu.async_copy(buf, agg_sh.at[dst_v.at[j]], sem, add=True)

        def scatter_wait(j, buf, sem):
            pltpu.make_async_copy(buf, agg_sh.at[dst_v.at[j]], sem).wait()

        def blk(b, carry):
            off = sid * CPT + b * IB
            pltpu.sync_copy(srcg_hbm.at[pl.ds(off, IB)], src_v)
            pltpu.sync_copy(dstg_hbm.at[pl.ds(off, IB)], dst_v)
            gather_start(0, rows_a, sem_a)

            def body(k2, c):
                j = 2 * k2
                gather_start(j + 1, rows_b, sem_b)
                gather_wait(j, rows_a, sem_a)
                pltpu.sync_copy(rows_a, agg_sh.at[dst_v.at[j]], add=True)

                @pl.when(j + 2 < IB)
                def _():
                    gather_start(j + 2, rows_a, sem_a)

                gather_wait(j + 1, rows_b, sem_b)
                pltpu.sync_copy(rows_b, agg_sh.at[dst_v.at[j + 1]],
                                add=True)
                return c

            lax.fori_loop(0, IB // 2, body, 0)
            return carry

        lax.fori_loop(0, NBLK, blk, 0)
        plsc.subcore_barrier()

        @pl.when(cid == 0)
        def _():
            pltpu.sync_copy(agg_sh.at[pl.ds(sid * WR, WR)],
                            out0_hbm.at[pl.ds(sid * WR, WR)])

        @pl.when(cid == 1)
        def _():
            pltpu.sync_copy(agg_sh.at[pl.ds(sid * WR, WR)],
                            out1_hbm.at[pl.ds(sid * WR, WR)])

    return agg_kernel(f0, f1, src_g, dst_g, zeros_agg)


def _tc_enc_a(h, enc_W1, b1):
    def body(h_ref, w_ref, b_ref, t_ref, stats_ref, acc_ref):
        i = pl.program_id(0)
        t = lax.dot_general(h_ref[...], w_ref[...], (((1,), (1,)), ((), ())),
                            preferred_element_type=jnp.float32) + b_ref[...]
        t_ref[...] = t
        s = jnp.concatenate([jnp.sum(t, 0, keepdims=True),
                             jnp.sum(t * t, 0, keepdims=True)], 0)

        @pl.when(i == 0)
        def _():
            acc_ref[...] = jnp.zeros_like(acc_ref)

        acc_ref[...] += s
        stats_ref[...] = acc_ref[...]

    return pl.pallas_call(
        body,
        grid=(GRID,),
        in_specs=[
            pl.BlockSpec((RB, IN_DIM), lambda i: (i, 0)),
            pl.BlockSpec((HID, IN_DIM), lambda i: (0, 0)),
            pl.BlockSpec((1, HID), lambda i: (0, 0)),
        ],
        out_specs=[
            pl.BlockSpec((RB, HID), lambda i: (i, 0)),
            pl.BlockSpec((2, HID), lambda i: (0, 0)),
        ],
        out_shape=[
            jax.ShapeDtypeStruct((N, HID), jnp.float32),
            jax.ShapeDtypeStruct((2, HID), jnp.float32),
        ],
        scratch_shapes=[pltpu.VMEM((2, HID), jnp.float32)],
    )(h, enc_W1, b1)


def _tc_enc_b(t, stats, gamma, beta, enc_W2, b2, deg_s):
    def body(t_ref, st_ref, g_ref, be_ref, w_ref, b_ref, ds_ref,
             x_ref, f0_ref, f1_ref):
        s = st_ref[...]
        mean = s[0:1, :] * (1.0 / N)
        var = s[1:2, :] * (1.0 / N) - mean * mean
        inv = lax.rsqrt(var + 1e-5)
        xn = (t_ref[...] - mean) * (inv * g_ref[...]) + be_ref[...]
        xn = jnp.maximum(xn, 0.0)
        x = lax.dot_general(xn, w_ref[...], (((1,), (1,)), ((), ())),
                            preferred_element_type=jnp.float32) + b_ref[...]
        x_ref[...] = x
        ns = lax.rsqrt(jnp.maximum(ds_ref[...][:, 0:1], 1.0))
        f = x * ns
        f0_ref[...] = f[:, : HID // 2]
        f1_ref[...] = f[:, HID // 2:]

    return pl.pallas_call(
        body,
        grid=(GRID,),
        in_specs=[
            pl.BlockSpec((RB, HID), lambda i: (i, 0)),
            pl.BlockSpec((2, HID), lambda i: (0, 0)),
            pl.BlockSpec((1, HID), lambda i: (0, 0)),
            pl.BlockSpec((1, HID), lambda i: (0, 0)),
            pl.BlockSpec((HID, HID), lambda i: (0, 0)),
            pl.BlockSpec((1, HID), lambda i: (0, 0)),
            pl.BlockSpec((RB, DEGC), lambda i: (i, 0)),
        ],
        out_specs=[
            pl.BlockSpec((RB, HID), lambda i: (i, 0)),
            pl.BlockSpec((RB, HID // 2), lambda i: (i, 0)),
            pl.BlockSpec((RB, HID // 2), lambda i: (i, 0)),
        ],
        out_shape=[
            jax.ShapeDtypeStruct((N, HID), jnp.float32),
            jax.ShapeDtypeStruct((SROWS, HID // 2), jnp.float32),
            jax.ShapeDtypeStruct((SROWS, HID // 2), jnp.float32),
        ],
    )(t, stats, gamma, beta, enc_W2, b2, deg_s)


def _tc_layer(a0, a1, deg_d, deg_s, W, acc, last):
    def body(a0_ref, a1_ref, dd_ref, ds_ref, w_ref, acc_ref, *out_refs):
        agg = jnp.concatenate([a0_ref[...], a1_ref[...]], 1)
        nd = lax.rsqrt(jnp.maximum(dd_ref[...][:, 0:1], 1.0))
        hc = lax.dot_general(agg * nd, w_ref[...], (((1,), (0,)), ((), ())),
                             preferred_element_type=jnp.float32)
        if not last:
            hc = jnp.maximum(hc, 0.0)
            out_refs[0][...] = acc_ref[...] + hc
            ns = lax.rsqrt(jnp.maximum(ds_ref[...][:, 0:1], 1.0))
            f = hc * ns
            out_refs[1][...] = f[:, : HID // 2]
            out_refs[2][...] = f[:, HID // 2:]
        else:
            out_refs[0][...] = (acc_ref[...] + hc) * (1.0 / (NUM_LAYERS + 1))

    out_specs = [pl.BlockSpec((RB, HID), lambda i: (i, 0))]
    out_shape = [jax.ShapeDtypeStruct((N, HID), jnp.float32)]
    if not last:
        out_specs += [pl.BlockSpec((RB, HID // 2), lambda i: (i, 0))] * 2
        out_shape += [jax.ShapeDtypeStruct((SROWS, HID // 2), jnp.float32)] * 2

    return pl.pallas_call(
        body,
        grid=(GRID,),
        in_specs=[
            pl.BlockSpec((RB, HID // 2), lambda i: (i, 0)),
            pl.BlockSpec((RB, HID // 2), lambda i: (i, 0)),
            pl.BlockSpec((RB, DEGC), lambda i: (i, 0)),
            pl.BlockSpec((RB, DEGC), lambda i: (i, 0)),
            pl.BlockSpec((HID, HID), lambda i: (0, 0)),
            pl.BlockSpec((RB, HID), lambda i: (i, 0)),
        ],
        out_specs=out_specs,
        out_shape=out_shape,
    )(a0, a1, deg_d, deg_s, W, acc)


def kernel(h, edge_index, enc_W1, enc_b1, bn_gamma, bn_beta, enc_W2,
           enc_b2, gcn_W):
    E = edge_index.shape[1]
    pad = EPAD - E
    src = edge_index[0]
    dst = edge_index[1]
    zero_pad = jnp.zeros((pad,), jnp.int32)
    sac_pad = jnp.full((pad,), N, jnp.int32)
    src_g = jnp.concatenate([src, zero_pad]).reshape(NCHUNKS, CHUNK)
    dst_g = jnp.concatenate([dst, sac_pad]).reshape(NCHUNKS, CHUNK)
    src_d = jnp.concatenate([src, sac_pad]).reshape(NCHUNKS, CHUNK)

    zeros_deg = jnp.zeros((ZR, DEGC), jnp.float32)
    ones_deg = jnp.ones((CHUNK, DEGC), jnp.float32)
    zeros_agg = jnp.zeros((ZR, HID // 2), jnp.float32)

    deg_s, deg_d = _sc_degrees(src_d, dst_g, zeros_deg, ones_deg)

    b1 = enc_b1.reshape(1, HID)
    b2 = enc_b2.reshape(1, HID)
    gamma = bn_gamma.reshape(1, HID)
    beta = bn_beta.reshape(1, HID)

    t, stats = _tc_enc_a(h, enc_W1, b1)
    acc, f0, f1 = _tc_enc_b(t, stats, gamma, beta, enc_W2, b2, deg_s)

    for i in range(NUM_LAYERS):
        a0, a1 = _sc_aggregate(f0, f1, src_g, dst_g, zeros_agg)
        last = i == NUM_LAYERS - 1
        outs = _tc_layer(a0, a1, deg_d, deg_s, gcn_W[i], acc, last)
        if last:
            acc = outs[0]
        else:
            acc, f0, f1 = outs
    return acc


# speedup vs baseline: 1.1580x; 1.0004x over previous
import functools

import jax
import jax.numpy as jnp
from jax import lax
from jax.experimental import pallas as pl
from jax.experimental.pallas import tpu as pltpu
from jax.experimental.pallas import tpu_sc as plsc

N = 50000
IN_DIM = 128
HID = 64
NUM_LAYERS = 5

CHUNK = 128
TILES = 16
CPT = 400
IB = 80
NBLK = CPT // IB
NCHUNKS = TILES * CPT
EPAD = NCHUNKS * CHUNK
SROWS = 50048
ZR = SROWS // TILES
WR = ZR
DEGC = 8

RB = 2176
FROWS = SROWS // 4
FB = RB // 4
GRID = SROWS // RB


def _sc_mesh():
    return plsc.VectorSubcoreMesh(core_axis_name="c", subcore_axis_name="s")


def _sc_degrees(src_d, dst_g, zeros_deg, ones_deg):
    @functools.partial(
        pl.kernel,
        mesh=_sc_mesh(),
        compiler_params=pltpu.CompilerParams(use_tc_tiling_on_sc=False),
        out_type=[
            jax.ShapeDtypeStruct((SROWS, DEGC), jnp.float32),
            jax.ShapeDtypeStruct((SROWS, DEGC), jnp.float32),
        ],
        scratch_types=[
            pltpu.VMEM((IB, CHUNK), jnp.int32),
            pltpu.VMEM((CHUNK, DEGC), jnp.float32),
            pltpu.VMEM_SHARED((SROWS, DEGC), jnp.float32),
            pltpu.SemaphoreType.DMA,
        ],
    )
    def deg_kernel(srcd_hbm, dstg_hbm, zeros_hbm, ones_hbm,
                   dego_hbm, degi_hbm, idx_v, ones_v, deg_sh, dsem):
        cid = lax.axis_index("c")
        sid = lax.axis_index("s")
        pltpu.sync_copy(zeros_hbm, deg_sh.at[pl.ds(sid * ZR, ZR)])
        pltpu.sync_copy(ones_hbm, ones_v)
        plsc.subcore_barrier()

        def blk(b, carry):
            off = sid * CPT + b * IB

            @pl.when(cid == 0)
            def _():
                pltpu.sync_copy(srcd_hbm.at[pl.ds(off, IB)], idx_v)

            @pl.when(cid == 1)
            def _():
                pltpu.sync_copy(dstg_hbm.at[pl.ds(off, IB)], idx_v)

            def s_start(j):
                pltpu.async_copy(ones_v, deg_sh.at[idx_v.at[j]], dsem,
                                 add=True)

            def s_wait(j):
                pltpu.make_async_copy(ones_v, deg_sh.at[idx_v.at[j]],
                                      dsem).wait()

            for j in range(4):
                s_start(j)

            def body(k, c):
                s_start(k + 4)
                s_wait(k)
                return c

            lax.fori_loop(0, IB - 4, body, 0)
            for j in range(4):
                s_wait(IB - 4 + j)
            return carry

        lax.fori_loop(0, NBLK, blk, 0)
        plsc.subcore_barrier()

        @pl.when(cid == 0)
        def _():
            pltpu.sync_copy(deg_sh.at[pl.ds(sid * WR, WR)],
                            dego_hbm.at[pl.ds(sid * WR, WR)])

        @pl.when(cid == 1)
        def _():
            pltpu.sync_copy(deg_sh.at[pl.ds(sid * WR, WR)],
                            degi_hbm.at[pl.ds(sid * WR, WR)])

    return deg_kernel(src_d, dst_g, zeros_deg, ones_deg)


def _sc_aggregate(f0, f1, src_g, dst_g, zeros_agg):
    @functools.partial(
        pl.kernel,
        mesh=_sc_mesh(),
        compiler_params=pltpu.CompilerParams(use_tc_tiling_on_sc=False),
        out_type=[
            jax.ShapeDtypeStruct((SROWS, HID // 2), jnp.float32),
            jax.ShapeDtypeStruct((SROWS, HID // 2), jnp.float32),
        ],
        scratch_types=[
            pltpu.VMEM((IB, CHUNK), jnp.int32),
            pltpu.VMEM((IB, CHUNK), jnp.int32),
            pltpu.VMEM((CHUNK, HID // 2), jnp.float32),
            pltpu.VMEM((CHUNK, HID // 2), jnp.float32),
            pltpu.VMEM_SHARED((SROWS, HID // 2), jnp.float32),
            pltpu.SemaphoreType.DMA,
            pltpu.SemaphoreType.DMA,
            pltpu.SemaphoreType.DMA,
            pltpu.SemaphoreType.DMA,
        ],
    )
    def agg_kernel(f0_hbm, f1_hbm, srcg_hbm, dstg_hbm, zeros_hbm,
                   out0_hbm, out1_hbm, src_v, dst_v, rows_a, rows_b,
                   agg_sh, sem_a, sem_b, ssem_a, ssem_b):
        cid = lax.axis_index("c")
        sid = lax.axis_index("s")
        pltpu.sync_copy(zeros_hbm, agg_sh.at[pl.ds(sid * ZR, ZR)])
        plsc.subcore_barrier()

        def gather_start(j, buf, sem):
            @pl.when(cid == 0)
            def _():
                pltpu.async_copy(f0_hbm.at[src_v.at[j]], buf, sem)

            @pl.when(cid == 1)
            def _():
                pltpu.async_copy(f1_hbm.at[src_v.at[j]], buf, sem)

        def gather_wait(j, buf, sem):
            pltpu.make_async_copy(f0_hbm.at[src_v.at[j]], buf, sem).wait()

        def scatter_start(j, buf, sem):
            pltpu.async_copy(buf, agg_sh.at[dst_v.at[j]], sem, add=True)

        def scatter_wait(j, buf, sem):
            pltpu.make_async_copy(buf, agg_sh.at[dst_v.at[j]], sem).wait()

        def blk(b, carry):
            off = sid * CPT + b * IB
            pltpu.sync_copy(srcg_hbm.at[pl.ds(off, IB)], src_v)
            pltpu.sync_copy(dstg_hbm.at[pl.ds(off, IB)], dst_v)
            gather_start(0, rows_a, sem_a)

            def body(k2, c):
                j = 2 * k2
                gather_start(j + 1, rows_b, sem_b)
                gather_wait(j, rows_a, sem_a)
                pltpu.sync_copy(rows_a, agg_sh.at[dst_v.at[j]], add=True)

                @pl.when(j + 2 < IB)
                def _():
                    gather_start(j + 2, rows_a, sem_a)

                gather_wait(j + 1, rows_b, sem_b)
                pltpu.sync_copy(rows_b, agg_sh.at[dst_v.at[j + 1]],
                                add=True)
                return c

            lax.fori_loop(0, IB // 2, body, 0)
            return carry

        lax.fori_loop(0, NBLK, blk, 0)
        plsc.subcore_barrier()

        @pl.when(cid == 0)
        def _():
            pltpu.sync_copy(agg_sh.at[pl.ds(sid * WR, WR)],
                            out0_hbm.at[pl.ds(sid * WR, WR)])

        @pl.when(cid == 1)
        def _():
            pltpu.sync_copy(agg_sh.at[pl.ds(sid * WR, WR)],
                            out1_hbm.at[pl.ds(sid * WR, WR)])

    return agg_kernel(f0, f1, src_g, dst_g, zeros_agg)


def _tc_enc_a(h, enc_W1, b1):
    def body(h_ref, w_ref, b_ref, t_ref, stats_ref, acc_ref):
        i = pl.program_id(0)
        t = lax.dot_general(h_ref[...], w_ref[...], (((1,), (1,)), ((), ())),
                            preferred_element_type=jnp.float32) + b_ref[...]
        t_ref[...] = t
        rows = lax.broadcasted_iota(jnp.int32, (RB, 1), 0) + i * RB
        tm = t * (rows < N).astype(jnp.float32)
        s = jnp.concatenate([jnp.sum(tm, 0, keepdims=True),
                             jnp.sum(tm * t, 0, keepdims=True)], 0)

        @pl.when(i == 0)
        def _():
            acc_ref[...] = jnp.zeros_like(acc_ref)

        acc_ref[...] += s
        stats_ref[...] = acc_ref[...]

    return pl.pallas_call(
        body,
        grid=(GRID,),
        in_specs=[
            pl.BlockSpec((RB, IN_DIM), lambda i: (i, 0)),
            pl.BlockSpec((HID, IN_DIM), lambda i: (0, 0)),
            pl.BlockSpec((1, HID), lambda i: (0, 0)),
        ],
        out_specs=[
            pl.BlockSpec((RB, HID), lambda i: (i, 0)),
            pl.BlockSpec((2, HID), lambda i: (0, 0)),
        ],
        out_shape=[
            jax.ShapeDtypeStruct((SROWS, HID), jnp.float32),
            jax.ShapeDtypeStruct((2, HID), jnp.float32),
        ],
        scratch_shapes=[pltpu.VMEM((2, HID), jnp.float32)],
    )(h, enc_W1, b1)


def _tc_enc_b(t, stats, gamma, beta, enc_W2, b2, deg_s):
    def body(t_ref, st_ref, g_ref, be_ref, w_ref, b_ref, ds_ref,
             x_ref, f0_ref, f1_ref):
        s = st_ref[...]
        mean = s[0:1, :] * (1.0 / N)
        var = s[1:2, :] * (1.0 / N) - mean * mean
        inv = lax.rsqrt(var + 1e-5)
        xn = (t_ref[...] - mean) * (inv * g_ref[...]) + be_ref[...]
        xn = jnp.maximum(xn, 0.0)
        x = lax.dot_general(xn, w_ref[...], (((1,), (1,)), ((), ())),
                            preferred_element_type=jnp.float32) + b_ref[...]
        x_ref[...] = x
        ns = lax.rsqrt(jnp.maximum(ds_ref[...][:, 0:1], 1.0))
        f = x * ns
        f0_ref[...] = f[:, : HID // 2]
        f1_ref[...] = f[:, HID // 2:]

    return pl.pallas_call(
        body,
        grid=(GRID,),
        in_specs=[
            pl.BlockSpec((RB, HID), lambda i: (i, 0)),
            pl.BlockSpec((2, HID), lambda i: (0, 0)),
            pl.BlockSpec((1, HID), lambda i: (0, 0)),
            pl.BlockSpec((1, HID), lambda i: (0, 0)),
            pl.BlockSpec((HID, HID), lambda i: (0, 0)),
            pl.BlockSpec((1, HID), lambda i: (0, 0)),
            pl.BlockSpec((RB, DEGC), lambda i: (i, 0)),
        ],
        out_specs=[
            pl.BlockSpec((RB, HID), lambda i: (i, 0)),
            pl.BlockSpec((RB, HID // 2), lambda i: (i, 0)),
            pl.BlockSpec((RB, HID // 2), lambda i: (i, 0)),
        ],
        out_shape=[
            jax.ShapeDtypeStruct((SROWS, HID), jnp.float32),
            jax.ShapeDtypeStruct((SROWS, HID // 2), jnp.float32),
            jax.ShapeDtypeStruct((SROWS, HID // 2), jnp.float32),
        ],
    )(t, stats, gamma, beta, enc_W2, b2, deg_s)


def _tc_layer(a0, a1, deg_d, deg_s, W, acc, last):
    def body(a0_ref, a1_ref, dd_ref, ds_ref, w_ref, acc_ref, *out_refs):
        agg = jnp.concatenate([a0_ref[...], a1_ref[...]], 1)
        nd = lax.rsqrt(jnp.maximum(dd_ref[...][:, 0:1], 1.0))
        hc = lax.dot_general(agg * nd, w_ref[...], (((1,), (0,)), ((), ())),
                             preferred_element_type=jnp.float32)
        if not last:
            hc = jnp.maximum(hc, 0.0)
            out_refs[0][...] = acc_ref[...] + hc
            ns = lax.rsqrt(jnp.maximum(ds_ref[...][:, 0:1], 1.0))
            f = hc * ns
            out_refs[1][...] = f[:, : HID // 2]
            out_refs[2][...] = f[:, HID // 2:]
        else:
            out_refs[0][...] = (acc_ref[...] + hc) * (1.0 / (NUM_LAYERS + 1))

    out_specs = [pl.BlockSpec((RB, HID), lambda i: (i, 0))]
    out_shape = [jax.ShapeDtypeStruct((SROWS, HID), jnp.float32)]
    if not last:
        out_specs += [pl.BlockSpec((RB, HID // 2), lambda i: (i, 0))] * 2
        out_shape += [jax.ShapeDtypeStruct((SROWS, HID // 2), jnp.float32)] * 2

    return pl.pallas_call(
        body,
        grid=(GRID,),
        in_specs=[
            pl.BlockSpec((RB, HID // 2), lambda i: (i, 0)),
            pl.BlockSpec((RB, HID // 2), lambda i: (i, 0)),
            pl.BlockSpec((RB, DEGC), lambda i: (i, 0)),
            pl.BlockSpec((RB, DEGC), lambda i: (i, 0)),
            pl.BlockSpec((HID, HID), lambda i: (0, 0)),
            pl.BlockSpec((RB, HID), lambda i: (i, 0)),
        ],
        out_specs=out_specs,
        out_shape=out_shape,
    )(a0, a1, deg_d, deg_s, W, acc)


def kernel(h, edge_index, enc_W1, enc_b1, bn_gamma, bn_beta, enc_W2,
           enc_b2, gcn_W):
    E = edge_index.shape[1]
    pad = EPAD - E
    src = edge_index[0]
    dst = edge_index[1]
    zero_pad = jnp.zeros((pad,), jnp.int32)
    sac_pad = jnp.full((pad,), N, jnp.int32)
    src_g = jnp.concatenate([src, zero_pad]).reshape(NCHUNKS, CHUNK)
    dst_g = jnp.concatenate([dst, sac_pad]).reshape(NCHUNKS, CHUNK)
    src_d = jnp.concatenate([src, sac_pad]).reshape(NCHUNKS, CHUNK)

    zeros_deg = jnp.zeros((ZR, DEGC), jnp.float32)
    ones_deg = jnp.ones((CHUNK, DEGC), jnp.float32)
    zeros_agg = jnp.zeros((ZR, HID // 2), jnp.float32)

    deg_s, deg_d = _sc_degrees(src_d, dst_g, zeros_deg, ones_deg)

    b1 = enc_b1.reshape(1, HID)
    b2 = enc_b2.reshape(1, HID)
    gamma = bn_gamma.reshape(1, HID)
    beta = bn_beta.reshape(1, HID)

    h_pad = jnp.concatenate(
        [h, jnp.zeros((SROWS - N, IN_DIM), jnp.float32)], 0)
    t, stats = _tc_enc_a(h_pad, enc_W1, b1)
    acc, f0, f1 = _tc_enc_b(t, stats, gamma, beta, enc_W2, b2, deg_s)

    for i in range(NUM_LAYERS):
        a0, a1 = _sc_aggregate(f0, f1, src_g, dst_g, zeros_agg)
        last = i == NUM_LAYERS - 1
        outs = _tc_layer(a0, a1, deg_d, deg_s, gcn_W[i], acc, last)
        if last:
            acc = outs[0]
        else:
            acc, f0, f1 = outs
    return acc[:N]


# speedup vs baseline: 1.2424x; 1.0729x over previous
import functools

import jax
import jax.numpy as jnp
from jax import lax
from jax.experimental import pallas as pl
from jax.experimental.pallas import tpu as pltpu
from jax.experimental.pallas import tpu_sc as plsc

N = 50000
IN_DIM = 128
HID = 64
NUM_LAYERS = 5

CHUNK = 128
TILES = 16
CPT = 400
IB = 40
NBLK = CPT // IB
NCHUNKS = TILES * CPT
EPAD = NCHUNKS * CHUNK
SROWS = 50048
ZR = SROWS // TILES
WR = ZR
DEGC = 8

RB = 2176
FROWS = SROWS // 4
FB = RB // 4
GRID = SROWS // RB


def _sc_mesh():
    return plsc.VectorSubcoreMesh(core_axis_name="c", subcore_axis_name="s")


def _sc_degrees(src_d, dst_g, zeros_deg, ones_deg):
    @functools.partial(
        pl.kernel,
        mesh=_sc_mesh(),
        compiler_params=pltpu.CompilerParams(use_tc_tiling_on_sc=False),
        out_type=[
            jax.ShapeDtypeStruct((SROWS, DEGC), jnp.float32),
            jax.ShapeDtypeStruct((SROWS, DEGC), jnp.float32),
        ],
        scratch_types=[
            pltpu.VMEM((IB, CHUNK), jnp.int32),
            pltpu.VMEM((CHUNK, DEGC), jnp.float32),
            pltpu.VMEM_SHARED((SROWS, DEGC), jnp.float32),
            pltpu.SemaphoreType.DMA,
        ],
    )
    def deg_kernel(srcd_hbm, dstg_hbm, zeros_hbm, ones_hbm,
                   dego_hbm, degi_hbm, idx_v, ones_v, deg_sh, dsem):
        cid = lax.axis_index("c")
        sid = lax.axis_index("s")
        pltpu.sync_copy(zeros_hbm, deg_sh.at[pl.ds(sid * ZR, ZR)])
        pltpu.sync_copy(ones_hbm, ones_v)
        plsc.subcore_barrier()

        def blk(b, carry):
            off = sid * CPT + b * IB

            @pl.when(cid == 0)
            def _():
                pltpu.sync_copy(srcd_hbm.at[pl.ds(off, IB)], idx_v)

            @pl.when(cid == 1)
            def _():
                pltpu.sync_copy(dstg_hbm.at[pl.ds(off, IB)], idx_v)

            def s_start(j):
                pltpu.async_copy(ones_v, deg_sh.at[idx_v.at[j]], dsem,
                                 add=True)

            def s_wait(j):
                pltpu.make_async_copy(ones_v, deg_sh.at[idx_v.at[j]],
                                      dsem).wait()

            for j in range(4):
                s_start(j)

            def body(k, c):
                s_start(k + 4)
                s_wait(k)
                return c

            lax.fori_loop(0, IB - 4, body, 0)
            for j in range(4):
                s_wait(IB - 4 + j)
            return carry

        lax.fori_loop(0, NBLK, blk, 0)
        plsc.subcore_barrier()

        @pl.when(cid == 0)
        def _():
            pltpu.sync_copy(deg_sh.at[pl.ds(sid * WR, WR)],
                            dego_hbm.at[pl.ds(sid * WR, WR)])

        @pl.when(cid == 1)
        def _():
            pltpu.sync_copy(deg_sh.at[pl.ds(sid * WR, WR)],
                            degi_hbm.at[pl.ds(sid * WR, WR)])

    return deg_kernel(src_d, dst_g, zeros_deg, ones_deg)


def _sc_aggregate(f0, f1, src_g, dst_g, zeros_agg):
    @functools.partial(
        pl.kernel,
        mesh=_sc_mesh(),
        compiler_params=pltpu.CompilerParams(use_tc_tiling_on_sc=False),
        out_type=[
            jax.ShapeDtypeStruct((SROWS, HID // 2), jnp.float32),
            jax.ShapeDtypeStruct((SROWS, HID // 2), jnp.float32),
        ],
        scratch_types=[
            pltpu.VMEM((IB, CHUNK), jnp.int32),
            pltpu.VMEM((IB, CHUNK), jnp.int32),
            pltpu.VMEM((CHUNK, HID // 2), jnp.float32),
            pltpu.VMEM((CHUNK, HID // 2), jnp.float32),
            pltpu.VMEM((CHUNK, HID // 2), jnp.float32),
            pltpu.VMEM((CHUNK, HID // 2), jnp.float32),
            pltpu.SemaphoreType.DMA,
            pltpu.SemaphoreType.DMA,
            pltpu.SemaphoreType.DMA,
            pltpu.SemaphoreType.DMA,
            pltpu.VMEM_SHARED((SROWS, HID // 2), jnp.float32),
        ],
    )
    def agg_kernel(f0_hbm, f1_hbm, srcg_hbm, dstg_hbm, zeros_hbm,
                   out0_hbm, out1_hbm, src_v, dst_v, rows_a, rows_b,
                   rows_c, rows_d, sem_a, sem_b, sem_c, sem_d, agg_sh):
        cid = lax.axis_index("c")
        sid = lax.axis_index("s")
        pltpu.sync_copy(zeros_hbm, agg_sh.at[pl.ds(sid * ZR, ZR)])
        plsc.subcore_barrier()

        def gather_start(j, buf, sem):
            @pl.when(cid == 0)
            def _():
                pltpu.async_copy(f0_hbm.at[src_v.at[j]], buf, sem)

            @pl.when(cid == 1)
            def _():
                pltpu.async_copy(f1_hbm.at[src_v.at[j]], buf, sem)

        def gather_wait(j, buf, sem):
            pltpu.make_async_copy(f0_hbm.at[src_v.at[j]], buf, sem).wait()

        bufs = ((rows_a, sem_a), (rows_b, sem_b),
                (rows_c, sem_c), (rows_d, sem_d))

        def blk(b, carry):
            off = sid * CPT + b * IB
            pltpu.sync_copy(srcg_hbm.at[pl.ds(off, IB)], src_v)
            pltpu.sync_copy(dstg_hbm.at[pl.ds(off, IB)], dst_v)
            for i in range(3):
                gather_start(i, *bufs[i])

            def body(k4, c):
                j = 4 * k4
                for i in range(4):
                    nxt = j + i + 3
                    buf, sem = bufs[(i + 3) % 4]

                    @pl.when(nxt < IB)
                    def _():
                        gather_start(nxt, buf, sem)

                    gather_wait(j + i, *bufs[i])
                    pltpu.sync_copy(bufs[i][0], agg_sh.at[dst_v.at[j + i]],
                                    add=True)
                return c

            lax.fori_loop(0, IB // 4, body, 0)
            return carry

        lax.fori_loop(0, NBLK, blk, 0)
        plsc.subcore_barrier()

        @pl.when(cid == 0)
        def _():
            pltpu.sync_copy(agg_sh.at[pl.ds(sid * WR, WR)],
                            out0_hbm.at[pl.ds(sid * WR, WR)])

        @pl.when(cid == 1)
        def _():
            pltpu.sync_copy(agg_sh.at[pl.ds(sid * WR, WR)],
                            out1_hbm.at[pl.ds(sid * WR, WR)])

    return agg_kernel(f0, f1, src_g, dst_g, zeros_agg)


def _tc_enc_a(h, enc_W1, b1):
    def body(h_ref, w_ref, b_ref, t_ref, stats_ref, acc_ref):
        i = pl.program_id(0)
        t = lax.dot_general(h_ref[...], w_ref[...], (((1,), (1,)), ((), ())),
                            preferred_element_type=jnp.float32) + b_ref[...]
        t_ref[...] = t
        rows = lax.broadcasted_iota(jnp.int32, (RB, 1), 0) + i * RB
        tm = t * (rows < N).astype(jnp.float32)
        s = jnp.concatenate([jnp.sum(tm, 0, keepdims=True),
                             jnp.sum(tm * t, 0, keepdims=True)], 0)

        @pl.when(i == 0)
        def _():
            acc_ref[...] = jnp.zeros_like(acc_ref)

        acc_ref[...] += s
        stats_ref[...] = acc_ref[...]

    return pl.pallas_call(
        body,
        grid=(GRID,),
        in_specs=[
            pl.BlockSpec((RB, IN_DIM), lambda i: (i, 0)),
            pl.BlockSpec((HID, IN_DIM), lambda i: (0, 0)),
            pl.BlockSpec((1, HID), lambda i: (0, 0)),
        ],
        out_specs=[
            pl.BlockSpec((RB, HID), lambda i: (i, 0)),
            pl.BlockSpec((2, HID), lambda i: (0, 0)),
        ],
        out_shape=[
            jax.ShapeDtypeStruct((SROWS, HID), jnp.float32),
            jax.ShapeDtypeStruct((2, HID), jnp.float32),
        ],
        scratch_shapes=[pltpu.VMEM((2, HID), jnp.float32)],
    )(h, enc_W1, b1)


def _tc_enc_b(t, stats, gamma, beta, enc_W2, b2, deg_s):
    def body(t_ref, st_ref, g_ref, be_ref, w_ref, b_ref, ds_ref,
             x_ref, f0_ref, f1_ref):
        s = st_ref[...]
        mean = s[0:1, :] * (1.0 / N)
        var = s[1:2, :] * (1.0 / N) - mean * mean
        inv = lax.rsqrt(var + 1e-5)
        xn = (t_ref[...] - mean) * (inv * g_ref[...]) + be_ref[...]
        xn = jnp.maximum(xn, 0.0)
        x = lax.dot_general(xn, w_ref[...], (((1,), (1,)), ((), ())),
                            preferred_element_type=jnp.float32) + b_ref[...]
        x_ref[...] = x
        ns = lax.rsqrt(jnp.maximum(ds_ref[...][:, 0:1], 1.0))
        f = x * ns
        f0_ref[...] = f[:, : HID // 2]
        f1_ref[...] = f[:, HID // 2:]

    return pl.pallas_call(
        body,
        grid=(GRID,),
        in_specs=[
            pl.BlockSpec((RB, HID), lambda i: (i, 0)),
            pl.BlockSpec((2, HID), lambda i: (0, 0)),
            pl.BlockSpec((1, HID), lambda i: (0, 0)),
            pl.BlockSpec((1, HID), lambda i: (0, 0)),
            pl.BlockSpec((HID, HID), lambda i: (0, 0)),
            pl.BlockSpec((1, HID), lambda i: (0, 0)),
            pl.BlockSpec((RB, DEGC), lambda i: (i, 0)),
        ],
        out_specs=[
            pl.BlockSpec((RB, HID), lambda i: (i, 0)),
            pl.BlockSpec((RB, HID // 2), lambda i: (i, 0)),
            pl.BlockSpec((RB, HID // 2), lambda i: (i, 0)),
        ],
        out_shape=[
            jax.ShapeDtypeStruct((SROWS, HID), jnp.float32),
            jax.ShapeDtypeStruct((SROWS, HID // 2), jnp.float32),
            jax.ShapeDtypeStruct((SROWS, HID // 2), jnp.float32),
        ],
    )(t, stats, gamma, beta, enc_W2, b2, deg_s)


def _tc_layer(a0, a1, deg_d, deg_s, W, acc, last):
    def body(a0_ref, a1_ref, dd_ref, ds_ref, w_ref, acc_ref, *out_refs):
        agg = jnp.concatenate([a0_ref[...], a1_ref[...]], 1)
        nd = lax.rsqrt(jnp.maximum(dd_ref[...][:, 0:1], 1.0))
        hc = lax.dot_general(agg * nd, w_ref[...], (((1,), (0,)), ((), ())),
                             preferred_element_type=jnp.float32)
        if not last:
            hc = jnp.maximum(hc, 0.0)
            out_refs[0][...] = acc_ref[...] + hc
            ns = lax.rsqrt(jnp.maximum(ds_ref[...][:, 0:1], 1.0))
            f = hc * ns
            out_refs[1][...] = f[:, : HID // 2]
            out_refs[2][...] = f[:, HID // 2:]
        else:
            out_refs[0][...] = (acc_ref[...] + hc) * (1.0 / (NUM_LAYERS + 1))

    out_specs = [pl.BlockSpec((RB, HID), lambda i: (i, 0))]
    out_shape = [jax.ShapeDtypeStruct((SROWS, HID), jnp.float32)]
    if not last:
        out_specs += [pl.BlockSpec((RB, HID // 2), lambda i: (i, 0))] * 2
        out_shape += [jax.ShapeDtypeStruct((SROWS, HID // 2), jnp.float32)] * 2

    return pl.pallas_call(
        body,
        grid=(GRID,),
        in_specs=[
            pl.BlockSpec((RB, HID // 2), lambda i: (i, 0)),
            pl.BlockSpec((RB, HID // 2), lambda i: (i, 0)),
            pl.BlockSpec((RB, DEGC), lambda i: (i, 0)),
            pl.BlockSpec((RB, DEGC), lambda i: (i, 0)),
            pl.BlockSpec((HID, HID), lambda i: (0, 0)),
            pl.BlockSpec((RB, HID), lambda i: (i, 0)),
        ],
        out_specs=out_specs,
        out_shape=out_shape,
    )(a0, a1, deg_d, deg_s, W, acc)


def kernel(h, edge_index, enc_W1, enc_b1, bn_gamma, bn_beta, enc_W2,
           enc_b2, gcn_W):
    E = edge_index.shape[1]
    pad = EPAD - E
    src = edge_index[0]
    dst = edge_index[1]
    zero_pad = jnp.zeros((pad,), jnp.int32)
    sac_pad = jnp.full((pad,), N, jnp.int32)
    src_g = jnp.concatenate([src, zero_pad]).reshape(NCHUNKS, CHUNK)
    dst_g = jnp.concatenate([dst, sac_pad]).reshape(NCHUNKS, CHUNK)
    src_d = jnp.concatenate([src, sac_pad]).reshape(NCHUNKS, CHUNK)

    zeros_deg = jnp.zeros((ZR, DEGC), jnp.float32)
    ones_deg = jnp.ones((CHUNK, DEGC), jnp.float32)
    zeros_agg = jnp.zeros((ZR, HID // 2), jnp.float32)

    deg_s, deg_d = _sc_degrees(src_d, dst_g, zeros_deg, ones_deg)

    b1 = enc_b1.reshape(1, HID)
    b2 = enc_b2.reshape(1, HID)
    gamma = bn_gamma.reshape(1, HID)
    beta = bn_beta.reshape(1, HID)

    h_pad = jnp.concatenate(
        [h, jnp.zeros((SROWS - N, IN_DIM), jnp.float32)], 0)
    t, stats = _tc_enc_a(h_pad, enc_W1, b1)
    acc, f0, f1 = _tc_enc_b(t, stats, gamma, beta, enc_W2, b2, deg_s)

    for i in range(NUM_LAYERS):
        a0, a1 = _sc_aggregate(f0, f1, src_g, dst_g, zeros_agg)
        last = i == NUM_LAYERS - 1
        outs = _tc_layer(a0, a1, deg_d, deg_s, gcn_W[i], acc, last)
        if last:
            acc = outs[0]
        else:
            acc, f0, f1 = outs
    return acc[:N]
